# Initial kernel scaffold; baseline (speedup 1.0000x reference)
import jax
import jax.numpy as jnp
from jax.experimental import pallas as pl


def kernel(x, edge_index, batch, Wl1, bl1, Wr1, Wl2, bl2, Wr2, Wl3, bl3, Wr3, Wl4, bl4, Wr4, W_fc1, b_fc1, W_fc2, b_fc2):
    raise NotImplementedError("write your pallas kernel here")


# speedup vs baseline: 4.4312x; 4.4312x over previous
import functools

import jax
import jax.numpy as jnp
from jax import lax
from jax.experimental import pallas as pl
from jax.experimental.pallas import tpu as pltpu
from jax.experimental.pallas import tpu_sc as plsc

N = 10000
E = 320000
G = 16
NC, NS = 2, 16
NW = NC * NS
EPW = E // NW
CH = 80
NCHUNK = EPW // CH
NPAD = 10240
RPT = NPAD // NS
RB = 1000
NRB = N // RB


@functools.lru_cache(maxsize=None)
def _sc_segsum(dpad):
  mesh = plsc.VectorSubcoreMesh(core_axis_name="c", subcore_axis_name="s")

  @functools.partial(
      pl.kernel,
      out_type=jax.ShapeDtypeStruct((NC, NPAD, dpad), jnp.float32),
      mesh=mesh,
      compiler_params=pltpu.CompilerParams(use_tc_tiling_on_sc=False),
      scratch_types=[
          pltpu.VMEM((RPT, dpad), jnp.float32),
          pltpu.VMEM((CH, dpad), jnp.float32),
          pltpu.VMEM((CH,), jnp.int32),
          pltpu.VMEM((CH,), jnp.int32),
          pltpu.VMEM_SHARED((NPAD, dpad), jnp.float32),
          pltpu.SemaphoreType.DMA,
      ],
  )
  def k(y_hbm, src_hbm, dst_hbm, out_hbm, zbuf, rows, srcv, dstv, acc, sem):
    cid = lax.axis_index("c")
    sid = lax.axis_index("s")
    wid = cid * NS + sid

    def zrow(r, carry):
      for j in range(dpad // 16):
        zbuf[r, pl.ds(j * 16, 16)] = jnp.zeros((16,), jnp.float32)
      return carry

    lax.fori_loop(0, RPT, zrow, 0)
    pltpu.sync_copy(zbuf, acc.at[pl.ds(sid * RPT, RPT)])
    plsc.subcore_barrier()

    base = wid * EPW

    def chunk(i, carry):
      off = base + i * CH
      pltpu.sync_copy(src_hbm.at[pl.ds(off, CH)], srcv)
      pltpu.sync_copy(dst_hbm.at[pl.ds(off, CH)], dstv)
      pltpu.async_copy(y_hbm.at[srcv], rows, sem).wait()
      pltpu.sync_copy(rows, acc.at[dstv], add=True)
      return carry

    lax.fori_loop(0, NCHUNK, chunk, 0)
    plsc.subcore_barrier()

    pltpu.sync_copy(acc.at[pl.ds(sid * RPT, RPT)], zbuf)
    pltpu.sync_copy(zbuf, out_hbm.at[cid, pl.ds(sid * RPT, RPT)])

  return k


def _full(shape):
  return pl.BlockSpec(shape, lambda i: tuple(0 for _ in shape))


def _rows(width):
  return pl.BlockSpec((RB, width), lambda i: (i, 0))


def _tc_pre(x, Wl1, Wr1):
  def body(x_ref, wl_ref, wr_ref, ypad_ref, r_ref):
    xb = x_ref[...]
    y = jnp.dot(xb, wl_ref[...], preferred_element_type=jnp.float32)
    pad = jnp.concatenate(
        [jnp.ones((RB, 1), jnp.float32), jnp.zeros((RB, 31), jnp.float32)], 1)
    ypad_ref[...] = jnp.concatenate([y, pad], 1)
    r_ref[...] = jnp.dot(xb, wr_ref[...], preferred_element_type=jnp.float32)

  return pl.pallas_call(
      body,
      grid=(NRB,),
      in_specs=[_rows(128), _full((128, 32)), _full((128, 32))],
      out_specs=[_rows(64), _rows(32)],
      out_shape=[
          jax.ShapeDtypeStruct((N, 64), jnp.float32),
          jax.ShapeDtypeStruct((N, 32), jnp.float32),
      ],
  )(x, Wl1, Wr1)


def _tc_combine1(p, r1, bl1):
  def body(p_ref, r_ref, bl_ref, h_ref, inv_ref):
    s = p_ref[0] + p_ref[1]
    inv = 1.0 / jnp.maximum(s[:, 32:33], 1.0)
    h = jnp.maximum(s[:, :32] * inv + bl_ref[...] + r_ref[...], 0.0)
    h_ref[...] = jnp.concatenate([h, jnp.zeros((RB, 32), jnp.float32)], 1)
    inv_ref[...] = inv

  return pl.pallas_call(
      body,
      grid=(NRB,),
      in_specs=[
          pl.BlockSpec((NC, RB, 64), lambda i: (0, i, 0)),
          _rows(32), _full((1, 32)),
      ],
      out_specs=[_rows(64), _rows(1)],
      out_shape=[
          jax.ShapeDtypeStruct((N, 64), jnp.float32),
          jax.ShapeDtypeStruct((N, 1), jnp.float32),
      ],
  )(p, r1, bl1)


def _tc_layer(p, invc, h_prev, Wl, bl, Wr, fi, fo, wp, hw):
  def body(p_ref, inv_ref, h_ref, wl_ref, bl_ref, wr_ref, o_ref):
    mean = (p_ref[0, :, :fi] + p_ref[1, :, :fi]) * inv_ref[...]
    o = jnp.dot(mean, wl_ref[...], preferred_element_type=jnp.float32)
    o += jnp.dot(h_ref[:, :fi], wr_ref[...],
                 preferred_element_type=jnp.float32)
    o_ref[...] = jnp.maximum(o + bl_ref[...], 0.0)

  if fo > 64:
    def body(p_ref, inv_ref, h_ref, wl_ref, bl_ref, wr_ref, *o_refs):
      mean = (p_ref[0, :, :fi] + p_ref[1, :, :fi]) * inv_ref[...]
      o = jnp.dot(mean, wl_ref[...], preferred_element_type=jnp.float32)
      o += jnp.dot(h_ref[:, :fi], wr_ref[...],
                   preferred_element_type=jnp.float32)
      o = jnp.maximum(o + bl_ref[...], 0.0)
      for j, o_ref in enumerate(o_refs):
        o_ref[...] = o[:, j * 64:(j + 1) * 64]

    nsp = fo // 64
    return pl.pallas_call(
        body,
        grid=(NRB,),
        in_specs=[
            pl.BlockSpec((NC, RB, wp), lambda i: (0, i, 0)),
            _rows(1), _rows(hw), _full((fi, fo)), _full((1, fo)),
            _full((fi, fo)),
        ],
        out_specs=[_rows(64)] * nsp,
        out_shape=[jax.ShapeDtypeStruct((N, 64), jnp.float32)] * nsp,
    )(p, invc, h_prev, Wl, bl, Wr)

  return pl.pallas_call(
      body,
      grid=(NRB,),
      in_specs=[
          pl.BlockSpec((NC, RB, wp), lambda i: (0, i, 0)),
          _rows(1), _rows(hw), _full((fi, fo)), _full((1, fo)),
          _full((fi, fo)),
      ],
      out_specs=_rows(fo),
      out_shape=jax.ShapeDtypeStruct((N, fo), jnp.float32),
  )(p, invc, h_prev, Wl, bl, Wr)


def _tc_final(pa, pb, invc, h3a, h3b, Wl4, bl4, Wr4, batch2, W1, b1, W2, b2):
  def body(pa_ref, pb_ref, inv_ref, ha_ref, hb_ref, wl_ref, bl_ref, wr_ref,
           b_ref, w1_ref, b1_ref, w2_ref, b2_ref, o_ref, acc):
    i = pl.program_id(0)
    inv = inv_ref[...]
    mean = jnp.concatenate(
        [(pa_ref[0] + pa_ref[1]) * inv, (pb_ref[0] + pb_ref[1]) * inv], 1)
    hb = jnp.concatenate([ha_ref[...], hb_ref[...]], 1)
    h4 = jnp.dot(mean, wl_ref[...], preferred_element_type=jnp.float32)
    h4 += jnp.dot(hb, wr_ref[...], preferred_element_type=jnp.float32)
    h4 = jnp.maximum(h4 + bl_ref[...], 0.0)
    oh = (b_ref[...] == lax.broadcasted_iota(jnp.int32, (1, G), 1))
    oh = oh.astype(jnp.float32)
    h4a = jnp.concatenate([h4, jnp.ones((RB, 16), jnp.float32)], 1)
    contrib = lax.dot_general(oh, h4a, (((0,), (0,)), ((), ())),
                              preferred_element_type=jnp.float32)

    @pl.when(i == 0)
    def _():
      acc[...] = contrib

    @pl.when(i > 0)
    def _():
      acc[...] += contrib

    @pl.when(i == NRB - 1)
    def _():
      pooled = acc[:, :256] / jnp.maximum(acc[:, 256:257], 1.0)
      o1 = jnp.dot(pooled, w1_ref[...], preferred_element_type=jnp.float32)
      o1 += b1_ref[...]
      o2 = jnp.dot(o1, w2_ref[...], preferred_element_type=jnp.float32)
      o2 += b2_ref[...]
      m = jnp.max(o2, axis=1, keepdims=True)
      lse = m + jnp.log(jnp.sum(jnp.exp(o2 - m), axis=1, keepdims=True))
      o_ref[...] = o2 - lse

  return pl.pallas_call(
      body,
      grid=(NRB,),
      in_specs=[
          pl.BlockSpec((NC, RB, 64), lambda i: (0, i, 0)),
          pl.BlockSpec((NC, RB, 64), lambda i: (0, i, 0)),
          _rows(1), _rows(64), _rows(64), _full((128, 256)), _full((1, 256)),
          _full((128, 256)), _rows(1), _full((256, 128)), _full((1, 128)),
          _full((128, 32)), _full((1, 32)),
      ],
      out_specs=_full((G, 32)),
      out_shape=jax.ShapeDtypeStruct((G, 32), jnp.float32),
      scratch_shapes=[pltpu.VMEM((G, 272), jnp.float32)],
  )(pa, pb, invc, h3a, h3b, Wl4, bl4, Wr4, batch2, W1, b1, W2, b2)


def kernel(x, edge_index, batch, Wl1, bl1, Wr1, Wl2, bl2, Wr2, Wl3, bl3, Wr3,
           Wl4, bl4, Wr4, W_fc1, b_fc1, W_fc2, b_fc2):
  src = edge_index[0]
  dst = edge_index[1]
  batch2 = batch.reshape(N, 1)
  bl1r, bl2r, bl3r, bl4r = (b.reshape(1, -1) for b in (bl1, bl2, bl3, bl4))
  b1r = b_fc1.reshape(1, -1)
  b2r = b_fc2.reshape(1, -1)

  ypad, r1 = _tc_pre(x, Wl1, Wr1)
  p1 = _sc_segsum(64)(ypad, src, dst)
  h1, invc = _tc_combine1(p1, r1, bl1r)
  p2 = _sc_segsum(64)(h1, src, dst)
  h2 = _tc_layer(p2, invc, h1, Wl2, bl2r, Wr2, 32, 64, 64, 64)
  p3 = _sc_segsum(64)(h2, src, dst)
  h3a, h3b = _tc_layer(p3, invc, h2, Wl3, bl3r, Wr3, 64, 128, 64, 64)
  p4a = _sc_segsum(64)(h3a, src, dst)
  p4b = _sc_segsum(64)(h3b, src, dst)
  return _tc_final(p4a, p4b, invc, h3a, h3b, Wl4, bl4r, Wr4, batch2,
                   W_fc1, b1r, W_fc2, b2r)


# speedup vs baseline: 11.3742x; 2.5668x over previous
import functools

import jax
import jax.numpy as jnp
from jax import lax
from jax.experimental import pallas as pl
from jax.experimental.pallas import tpu as pltpu
from jax.experimental.pallas import tpu_sc as plsc

N = 10000
E = 320000
G = 16
NC, NS = 2, 16
NW = NC * NS
EPW = E // NW
CH = 80
NCHUNK = EPW // CH
NPAD = 10240
RPT = NPAD // NS
RB = 1000
NRB = N // RB


@functools.lru_cache(maxsize=None)
def _sc_segsum(dpad):
  mesh = plsc.VectorSubcoreMesh(core_axis_name="c", subcore_axis_name="s")

  @functools.partial(
      pl.kernel,
      out_type=jax.ShapeDtypeStruct((NC, NPAD, dpad), jnp.float32),
      mesh=mesh,
      compiler_params=pltpu.CompilerParams(use_tc_tiling_on_sc=False),
      scratch_types=[
          pltpu.VMEM((RPT, dpad), jnp.float32),
          pltpu.VMEM((CH, dpad), jnp.float32),
          pltpu.VMEM((CH, dpad), jnp.float32),
          pltpu.VMEM((NCHUNK, CH), jnp.int32),
          pltpu.VMEM((NCHUNK, CH), jnp.int32),
          pltpu.VMEM_SHARED((NPAD, dpad), jnp.float32),
          pltpu.SemaphoreType.DMA,
          pltpu.SemaphoreType.DMA,
      ],
  )
  def k(y_hbm, src_hbm, dst_hbm, out_hbm, zbuf, rows0, rows1, srcs, dsts,
        acc, sem0, sem1):
    cid = lax.axis_index("c")
    sid = lax.axis_index("s")
    wid = cid * NS + sid
    rows = (rows0, rows1)
    sems = (sem0, sem1)

    def zrow(r, carry):
      for j in range(dpad // 16):
        zbuf[r, pl.ds(j * 16, 16)] = jnp.zeros((16,), jnp.float32)
      return carry

    lax.fori_loop(0, RPT, zrow, 0)
    pltpu.sync_copy(zbuf, acc.at[pl.ds(sid * RPT, RPT)])

    cbase = wid * NCHUNK
    pltpu.sync_copy(src_hbm.at[pl.ds(cbase, NCHUNK)], srcs)
    pltpu.sync_copy(dst_hbm.at[pl.ds(cbase, NCHUNK)], dsts)
    plsc.subcore_barrier()

    pltpu.async_copy(y_hbm.at[srcs.at[0]], rows0, sem0)
    pltpu.async_copy(y_hbm.at[srcs.at[1]], rows1, sem1)

    def step(kk, carry):
      for b in range(2):
        c = 2 * kk + b
        pltpu.make_async_copy(y_hbm.at[srcs.at[c]], rows[b], sems[b]).wait()
        pltpu.sync_copy(rows[b], acc.at[dsts.at[c]], add=True)

        @pl.when(c + 2 < NCHUNK)
        def _():
          pltpu.async_copy(y_hbm.at[srcs.at[c + 2]], rows[b], sems[b])
      return carry

    lax.fori_loop(0, NCHUNK // 2, step, 0)
    if NCHUNK % 2:
      c = NCHUNK - 1
      pltpu.make_async_copy(y_hbm.at[srcs.at[c]], rows0, sem0).wait()
      pltpu.sync_copy(rows0, acc.at[dsts.at[c]], add=True)
    plsc.subcore_barrier()

    pltpu.sync_copy(acc.at[pl.ds(sid * RPT, RPT)], zbuf)
    pltpu.sync_copy(zbuf, out_hbm.at[cid, pl.ds(sid * RPT, RPT)])

  return k


def _full(shape):
  return pl.BlockSpec(shape, lambda i: tuple(0 for _ in shape))


def _rows(width):
  return pl.BlockSpec((RB, width), lambda i: (i, 0))


def _tc_pre(x, Wl1, Wr1):
  def body(x_ref, wl_ref, wr_ref, ypad_ref, r_ref):
    xb = x_ref[...]
    y = jnp.dot(xb, wl_ref[...], preferred_element_type=jnp.float32)
    pad = jnp.concatenate(
        [jnp.ones((RB, 1), jnp.float32), jnp.zeros((RB, 15), jnp.float32)], 1)
    ypad_ref[...] = jnp.concatenate([y, pad], 1)
    r_ref[...] = jnp.dot(xb, wr_ref[...], preferred_element_type=jnp.float32)

  return pl.pallas_call(
      body,
      grid=(NRB,),
      in_specs=[_rows(128), _full((128, 32)), _full((128, 32))],
      out_specs=[_rows(48), _rows(32)],
      out_shape=[
          jax.ShapeDtypeStruct((N, 48), jnp.float32),
          jax.ShapeDtypeStruct((N, 32), jnp.float32),
      ],
  )(x, Wl1, Wr1)


def _tc_combine1(p, r1, bl1):
  def body(p_ref, r_ref, bl_ref, h_ref, inv_ref):
    s = p_ref[0] + p_ref[1]
    inv = 1.0 / jnp.maximum(s[:, 32:33], 1.0)
    h_ref[...] = jnp.maximum(s[:, :32] * inv + bl_ref[...] + r_ref[...], 0.0)
    inv_ref[...] = inv

  return pl.pallas_call(
      body,
      grid=(NRB,),
      in_specs=[
          pl.BlockSpec((NC, RB, 48), lambda i: (0, i, 0)),
          _rows(32), _full((1, 32)),
      ],
      out_specs=[_rows(32), _rows(1)],
      out_shape=[
          jax.ShapeDtypeStruct((N, 32), jnp.float32),
          jax.ShapeDtypeStruct((N, 1), jnp.float32),
      ],
  )(p, r1, bl1)


def _tc_layer(p, invc, h_prev, Wl, bl, Wr, fi, fo, wp, hw):
  def body(p_ref, inv_ref, h_ref, wl_ref, bl_ref, wr_ref, o_ref):
    mean = (p_ref[0, :, :fi] + p_ref[1, :, :fi]) * inv_ref[...]
    o = jnp.dot(mean, wl_ref[...], preferred_element_type=jnp.float32)
    o += jnp.dot(h_ref[:, :fi], wr_ref[...],
                 preferred_element_type=jnp.float32)
    o_ref[...] = jnp.maximum(o + bl_ref[...], 0.0)

  if fo > 64:
    def body(p_ref, inv_ref, h_ref, wl_ref, bl_ref, wr_ref, *o_refs):
      mean = (p_ref[0, :, :fi] + p_ref[1, :, :fi]) * inv_ref[...]
      o = jnp.dot(mean, wl_ref[...], preferred_element_type=jnp.float32)
      o += jnp.dot(h_ref[:, :fi], wr_ref[...],
                   preferred_element_type=jnp.float32)
      o = jnp.maximum(o + bl_ref[...], 0.0)
      for j, o_ref in enumerate(o_refs):
        o_ref[...] = o[:, j * 64:(j + 1) * 64]

    nsp = fo // 64
    return pl.pallas_call(
        body,
        grid=(NRB,),
        in_specs=[
            pl.BlockSpec((NC, RB, wp), lambda i: (0, i, 0)),
            _rows(1), _rows(hw), _full((fi, fo)), _full((1, fo)),
            _full((fi, fo)),
        ],
        out_specs=[_rows(64)] * nsp,
        out_shape=[jax.ShapeDtypeStruct((N, 64), jnp.float32)] * nsp,
    )(p, invc, h_prev, Wl, bl, Wr)

  return pl.pallas_call(
      body,
      grid=(NRB,),
      in_specs=[
          pl.BlockSpec((NC, RB, wp), lambda i: (0, i, 0)),
          _rows(1), _rows(hw), _full((fi, fo)), _full((1, fo)),
          _full((fi, fo)),
      ],
      out_specs=_rows(fo),
      out_shape=jax.ShapeDtypeStruct((N, fo), jnp.float32),
  )(p, invc, h_prev, Wl, bl, Wr)


def _tc_final(pa, pb, invc, h3a, h3b, Wl4, bl4, Wr4, batch2, W1, b1, W2, b2):
  def body(pa_ref, pb_ref, inv_ref, ha_ref, hb_ref, wl_ref, bl_ref, wr_ref,
           b_ref, w1_ref, b1_ref, w2_ref, b2_ref, o_ref, acc):
    i = pl.program_id(0)
    inv = inv_ref[...]
    mean = jnp.concatenate(
        [(pa_ref[0] + pa_ref[1]) * inv, (pb_ref[0] + pb_ref[1]) * inv], 1)
    hb = jnp.concatenate([ha_ref[...], hb_ref[...]], 1)
    h4 = jnp.dot(mean, wl_ref[...], preferred_element_type=jnp.float32)
    h4 += jnp.dot(hb, wr_ref[...], preferred_element_type=jnp.float32)
    h4 = jnp.maximum(h4 + bl_ref[...], 0.0)
    oh = (b_ref[...] == lax.broadcasted_iota(jnp.int32, (1, G), 1))
    oh = oh.astype(jnp.float32)
    h4a = jnp.concatenate([h4, jnp.ones((RB, 16), jnp.float32)], 1)
    contrib = lax.dot_general(oh, h4a, (((0,), (0,)), ((), ())),
                              preferred_element_type=jnp.float32)

    @pl.when(i == 0)
    def _():
      acc[...] = contrib

    @pl.when(i > 0)
    def _():
      acc[...] += contrib

    @pl.when(i == NRB - 1)
    def _():
      pooled = acc[:, :256] / jnp.maximum(acc[:, 256:257], 1.0)
      o1 = jnp.dot(pooled, w1_ref[...], preferred_element_type=jnp.float32)
      o1 += b1_ref[...]
      o2 = jnp.dot(o1, w2_ref[...], preferred_element_type=jnp.float32)
      o2 += b2_ref[...]
      m = jnp.max(o2, axis=1, keepdims=True)
      lse = m + jnp.log(jnp.sum(jnp.exp(o2 - m), axis=1, keepdims=True))
      o_ref[...] = o2 - lse

  return pl.pallas_call(
      body,
      grid=(NRB,),
      in_specs=[
          pl.BlockSpec((NC, RB, 64), lambda i: (0, i, 0)),
          pl.BlockSpec((NC, RB, 64), lambda i: (0, i, 0)),
          _rows(1), _rows(64), _rows(64), _full((128, 256)), _full((1, 256)),
          _full((128, 256)), _rows(1), _full((256, 128)), _full((1, 128)),
          _full((128, 32)), _full((1, 32)),
      ],
      out_specs=_full((G, 32)),
      out_shape=jax.ShapeDtypeStruct((G, 32), jnp.float32),
      scratch_shapes=[pltpu.VMEM((G, 272), jnp.float32)],
  )(pa, pb, invc, h3a, h3b, Wl4, bl4, Wr4, batch2, W1, b1, W2, b2)


def kernel(x, edge_index, batch, Wl1, bl1, Wr1, Wl2, bl2, Wr2, Wl3, bl3, Wr3,
           Wl4, bl4, Wr4, W_fc1, b_fc1, W_fc2, b_fc2):
  src = edge_index[0].reshape(E // CH, CH)
  dst = edge_index[1].reshape(E // CH, CH)
  batch2 = batch.reshape(N, 1)
  bl1r, bl2r, bl3r, bl4r = (b.reshape(1, -1) for b in (bl1, bl2, bl3, bl4))
  b1r = b_fc1.reshape(1, -1)
  b2r = b_fc2.reshape(1, -1)

  ypad, r1 = _tc_pre(x, Wl1, Wr1)
  p1 = _sc_segsum(48)(ypad, src, dst)
  h1, invc = _tc_combine1(p1, r1, bl1r)
  p2 = _sc_segsum(32)(h1, src, dst)
  h2 = _tc_layer(p2, invc, h1, Wl2, bl2r, Wr2, 32, 64, 32, 32)
  p3 = _sc_segsum(64)(h2, src, dst)
  h3a, h3b = _tc_layer(p3, invc, h2, Wl3, bl3r, Wr3, 64, 128, 64, 64)
  p4a = _sc_segsum(64)(h3a, src, dst)
  p4b = _sc_segsum(64)(h3b, src, dst)
  return _tc_final(p4a, p4b, invc, h3a, h3b, Wl4, bl4r, Wr4, batch2,
                   W_fc1, b1r, W_fc2, b2r)


# speedup vs baseline: 12.8908x; 1.1333x over previous
import functools

import jax
import jax.numpy as jnp
from jax import lax
from jax.experimental import pallas as pl
from jax.experimental.pallas import tpu as pltpu
from jax.experimental.pallas import tpu_sc as plsc

N = 10000
E = 320000
G = 16
NC, NS = 2, 16
NW = NC * NS
EPW = E // NW
CH = 125
NCHUNK = EPW // CH
NPAD = 10240
RPT = NPAD // NS
RB = 1000
NRB = N // RB


@functools.lru_cache(maxsize=None)
def _sc_segsum(dpad):
  mesh = plsc.VectorSubcoreMesh(core_axis_name="c", subcore_axis_name="s")

  @functools.partial(
      pl.kernel,
      out_type=jax.ShapeDtypeStruct((NC, NPAD, dpad), jnp.float32),
      mesh=mesh,
      compiler_params=pltpu.CompilerParams(use_tc_tiling_on_sc=False),
      scratch_types=[
          pltpu.VMEM((RPT, dpad), jnp.float32),
          pltpu.VMEM((CH, dpad), jnp.float32),
          pltpu.VMEM((CH, dpad), jnp.float32),
          pltpu.VMEM((NCHUNK, CH), jnp.int32),
          pltpu.VMEM((NCHUNK, CH), jnp.int32),
          pltpu.VMEM_SHARED((NPAD, dpad), jnp.float32),
          pltpu.SemaphoreType.DMA,
          pltpu.SemaphoreType.DMA,
      ],
  )
  def k(y_hbm, src_hbm, dst_hbm, out_hbm, zbuf, rows0, rows1, srcs, dsts,
        acc, sem0, sem1):
    cid = lax.axis_index("c")
    sid = lax.axis_index("s")
    wid = cid * NS + sid
    rows = (rows0, rows1)
    sems = (sem0, sem1)

    def zrow(r, carry):
      for j in range(dpad // 16):
        zbuf[r, pl.ds(j * 16, 16)] = jnp.zeros((16,), jnp.float32)
      return carry

    lax.fori_loop(0, RPT, zrow, 0)
    pltpu.sync_copy(zbuf, acc.at[pl.ds(sid * RPT, RPT)])

    cbase = wid * NCHUNK
    pltpu.sync_copy(src_hbm.at[pl.ds(cbase, NCHUNK)], srcs)
    pltpu.sync_copy(dst_hbm.at[pl.ds(cbase, NCHUNK)], dsts)
    plsc.subcore_barrier()

    pltpu.async_copy(y_hbm.at[srcs.at[0]], rows0, sem0)
    pltpu.async_copy(y_hbm.at[srcs.at[1]], rows1, sem1)

    def step(kk, carry):
      for b in range(2):
        c = 2 * kk + b
        pltpu.make_async_copy(y_hbm.at[srcs.at[c]], rows[b], sems[b]).wait()
        pltpu.sync_copy(rows[b], acc.at[dsts.at[c]], add=True)

        @pl.when(c + 2 < NCHUNK)
        def _():
          pltpu.async_copy(y_hbm.at[srcs.at[c + 2]], rows[b], sems[b])
      return carry

    lax.fori_loop(0, NCHUNK // 2, step, 0)
    plsc.subcore_barrier()

    pltpu.sync_copy(acc.at[pl.ds(sid * RPT, RPT)], zbuf)
    pltpu.sync_copy(zbuf, out_hbm.at[cid, pl.ds(sid * RPT, RPT)])

  return k


def _full(shape):
  return pl.BlockSpec(shape, lambda i: tuple(0 for _ in shape))


def _rows(width):
  return pl.BlockSpec((RB, width), lambda i: (i, 0))


def _tc_pre(x, Wl1, Wr1):
  def body(x_ref, wl_ref, wr_ref, ypad_ref, r_ref):
    xb = x_ref[...]
    y = jnp.dot(xb, wl_ref[...], preferred_element_type=jnp.float32)
    pad = jnp.concatenate(
        [jnp.ones((RB, 1), jnp.float32), jnp.zeros((RB, 15), jnp.float32)], 1)
    ypad_ref[...] = jnp.concatenate([y, pad], 1)
    r_ref[...] = jnp.dot(xb, wr_ref[...], preferred_element_type=jnp.float32)

  return pl.pallas_call(
      body,
      grid=(NRB,),
      in_specs=[_rows(128), _full((128, 32)), _full((128, 32))],
      out_specs=[_rows(48), _rows(32)],
      out_shape=[
          jax.ShapeDtypeStruct((N, 48), jnp.float32),
          jax.ShapeDtypeStruct((N, 32), jnp.float32),
      ],
  )(x, Wl1, Wr1)


def _tc_combine1(p, r1, bl1):
  def body(p_ref, r_ref, bl_ref, h_ref, inv_ref):
    s = p_ref[0] + p_ref[1]
    inv = 1.0 / jnp.maximum(s[:, 32:33], 1.0)
    h_ref[...] = jnp.maximum(s[:, :32] * inv + bl_ref[...] + r_ref[...], 0.0)
    inv_ref[...] = inv

  return pl.pallas_call(
      body,
      grid=(NRB,),
      in_specs=[
          pl.BlockSpec((NC, RB, 48), lambda i: (0, i, 0)),
          _rows(32), _full((1, 32)),
      ],
      out_specs=[_rows(32), _rows(1)],
      out_shape=[
          jax.ShapeDtypeStruct((N, 32), jnp.float32),
          jax.ShapeDtypeStruct((N, 1), jnp.float32),
      ],
  )(p, r1, bl1)


def _tc_layer(p, invc, h_prev, Wl, bl, Wr, fi, fo, wp, hw):
  def body(p_ref, inv_ref, h_ref, wl_ref, bl_ref, wr_ref, o_ref):
    mean = (p_ref[0, :, :fi] + p_ref[1, :, :fi]) * inv_ref[...]
    o = jnp.dot(mean, wl_ref[...], preferred_element_type=jnp.float32)
    o += jnp.dot(h_ref[:, :fi], wr_ref[...],
                 preferred_element_type=jnp.float32)
    o_ref[...] = jnp.maximum(o + bl_ref[...], 0.0)

  if fo > 64:
    def body(p_ref, inv_ref, h_ref, wl_ref, bl_ref, wr_ref, *o_refs):
      mean = (p_ref[0, :, :fi] + p_ref[1, :, :fi]) * inv_ref[...]
      o = jnp.dot(mean, wl_ref[...], preferred_element_type=jnp.float32)
      o += jnp.dot(h_ref[:, :fi], wr_ref[...],
                   preferred_element_type=jnp.float32)
      o = jnp.maximum(o + bl_ref[...], 0.0)
      for j, o_ref in enumerate(o_refs):
        o_ref[...] = o[:, j * 64:(j + 1) * 64]

    nsp = fo // 64
    return pl.pallas_call(
        body,
        grid=(NRB,),
        in_specs=[
            pl.BlockSpec((NC, RB, wp), lambda i: (0, i, 0)),
            _rows(1), _rows(hw), _full((fi, fo)), _full((1, fo)),
            _full((fi, fo)),
        ],
        out_specs=[_rows(64)] * nsp,
        out_shape=[jax.ShapeDtypeStruct((N, 64), jnp.float32)] * nsp,
    )(p, invc, h_prev, Wl, bl, Wr)

  return pl.pallas_call(
      body,
      grid=(NRB,),
      in_specs=[
          pl.BlockSpec((NC, RB, wp), lambda i: (0, i, 0)),
          _rows(1), _rows(hw), _full((fi, fo)), _full((1, fo)),
          _full((fi, fo)),
      ],
      out_specs=_rows(fo),
      out_shape=jax.ShapeDtypeStruct((N, fo), jnp.float32),
  )(p, invc, h_prev, Wl, bl, Wr)


def _tc_final(pa, pb, invc, h3a, h3b, Wl4, bl4, Wr4, batch2, W1, b1, W2, b2):
  def body(pa_ref, pb_ref, inv_ref, ha_ref, hb_ref, wl_ref, bl_ref, wr_ref,
           b_ref, w1_ref, b1_ref, w2_ref, b2_ref, o_ref, acc):
    i = pl.program_id(0)
    inv = inv_ref[...]
    mean = jnp.concatenate(
        [(pa_ref[0] + pa_ref[1]) * inv, (pb_ref[0] + pb_ref[1]) * inv], 1)
    hb = jnp.concatenate([ha_ref[...], hb_ref[...]], 1)
    h4 = jnp.dot(mean, wl_ref[...], preferred_element_type=jnp.float32)
    h4 += jnp.dot(hb, wr_ref[...], preferred_element_type=jnp.float32)
    h4 = jnp.maximum(h4 + bl_ref[...], 0.0)
    oh = (b_ref[...] == lax.broadcasted_iota(jnp.int32, (1, G), 1))
    oh = oh.astype(jnp.float32)
    h4a = jnp.concatenate([h4, jnp.ones((RB, 16), jnp.float32)], 1)
    contrib = lax.dot_general(oh, h4a, (((0,), (0,)), ((), ())),
                              preferred_element_type=jnp.float32)

    @pl.when(i == 0)
    def _():
      acc[...] = contrib

    @pl.when(i > 0)
    def _():
      acc[...] += contrib

    @pl.when(i == NRB - 1)
    def _():
      pooled = acc[:, :256] / jnp.maximum(acc[:, 256:257], 1.0)
      o1 = jnp.dot(pooled, w1_ref[...], preferred_element_type=jnp.float32)
      o1 += b1_ref[...]
      o2 = jnp.dot(o1, w2_ref[...], preferred_element_type=jnp.float32)
      o2 += b2_ref[...]
      m = jnp.max(o2, axis=1, keepdims=True)
      lse = m + jnp.log(jnp.sum(jnp.exp(o2 - m), axis=1, keepdims=True))
      o_ref[...] = o2 - lse

  return pl.pallas_call(
      body,
      grid=(NRB,),
      in_specs=[
          pl.BlockSpec((NC, RB, 64), lambda i: (0, i, 0)),
          pl.BlockSpec((NC, RB, 64), lambda i: (0, i, 0)),
          _rows(1), _rows(64), _rows(64), _full((128, 256)), _full((1, 256)),
          _full((128, 256)), _rows(1), _full((256, 128)), _full((1, 128)),
          _full((128, 32)), _full((1, 32)),
      ],
      out_specs=_full((G, 32)),
      out_shape=jax.ShapeDtypeStruct((G, 32), jnp.float32),
      scratch_shapes=[pltpu.VMEM((G, 272), jnp.float32)],
  )(pa, pb, invc, h3a, h3b, Wl4, bl4, Wr4, batch2, W1, b1, W2, b2)


def kernel(x, edge_index, batch, Wl1, bl1, Wr1, Wl2, bl2, Wr2, Wl3, bl3, Wr3,
           Wl4, bl4, Wr4, W_fc1, b_fc1, W_fc2, b_fc2):
  src = edge_index[0].reshape(E // CH, CH)
  dst = edge_index[1].reshape(E // CH, CH)
  batch2 = batch.reshape(N, 1)
  bl1r, bl2r, bl3r, bl4r = (b.reshape(1, -1) for b in (bl1, bl2, bl3, bl4))
  b1r = b_fc1.reshape(1, -1)
  b2r = b_fc2.reshape(1, -1)

  ypad, r1 = _tc_pre(x, Wl1, Wr1)
  p1 = _sc_segsum(48)(ypad, src, dst)
  h1, invc = _tc_combine1(p1, r1, bl1r)
  p2 = _sc_segsum(32)(h1, src, dst)
  h2 = _tc_layer(p2, invc, h1, Wl2, bl2r, Wr2, 32, 64, 32, 32)
  p3 = _sc_segsum(64)(h2, src, dst)
  h3a, h3b = _tc_layer(p3, invc, h2, Wl3, bl3r, Wr3, 64, 128, 64, 64)
  p4a = _sc_segsum(64)(h3a, src, dst)
  p4b = _sc_segsum(64)(h3b, src, dst)
  return _tc_final(p4a, p4b, invc, h3a, h3b, Wl4, bl4r, Wr4, batch2,
                   W_fc1, b1r, W_fc2, b2r)


# speedup vs baseline: 12.9668x; 1.0059x over previous
import functools

import jax
import jax.numpy as jnp
from jax import lax
from jax.experimental import pallas as pl
from jax.experimental.pallas import tpu as pltpu
from jax.experimental.pallas import tpu_sc as plsc

N = 10000
E = 320000
G = 16
NC, NS = 2, 16
NW = NC * NS
EPW = E // NW
CH = 125
NCHUNK = EPW // CH
NPAD = 10240
RPT = NPAD // NS
RB = 2000
NRB = N // RB


@functools.lru_cache(maxsize=None)
def _sc_segsum(dpad):
  mesh = plsc.VectorSubcoreMesh(core_axis_name="c", subcore_axis_name="s")

  @functools.partial(
      pl.kernel,
      out_type=jax.ShapeDtypeStruct((NC, NPAD, dpad), jnp.float32),
      mesh=mesh,
      compiler_params=pltpu.CompilerParams(use_tc_tiling_on_sc=False),
      scratch_types=[
          pltpu.VMEM((RPT, dpad), jnp.float32),
          pltpu.VMEM((CH, dpad), jnp.float32),
          pltpu.VMEM((CH, dpad), jnp.float32),
          pltpu.VMEM((NCHUNK, CH), jnp.int32),
          pltpu.VMEM((NCHUNK, CH), jnp.int32),
          pltpu.VMEM_SHARED((NPAD, dpad), jnp.float32),
          pltpu.SemaphoreType.DMA,
          pltpu.SemaphoreType.DMA,
      ],
  )
  def k(y_hbm, src_hbm, dst_hbm, out_hbm, zbuf, rows0, rows1, srcs, dsts,
        acc, sem0, sem1):
    cid = lax.axis_index("c")
    sid = lax.axis_index("s")
    wid = cid * NS + sid
    rows = (rows0, rows1)
    sems = (sem0, sem1)

    def zrow(r, carry):
      for j in range(dpad // 16):
        zbuf[r, pl.ds(j * 16, 16)] = jnp.zeros((16,), jnp.float32)
      return carry

    lax.fori_loop(0, RPT, zrow, 0)
    pltpu.sync_copy(zbuf, acc.at[pl.ds(sid * RPT, RPT)])

    cbase = wid * NCHUNK
    pltpu.sync_copy(src_hbm.at[pl.ds(cbase, NCHUNK)], srcs)
    pltpu.sync_copy(dst_hbm.at[pl.ds(cbase, NCHUNK)], dsts)
    plsc.subcore_barrier()

    pltpu.async_copy(y_hbm.at[srcs.at[0]], rows0, sem0)
    pltpu.async_copy(y_hbm.at[srcs.at[1]], rows1, sem1)

    def step(kk, carry):
      for b in range(2):
        c = 2 * kk + b
        pltpu.make_async_copy(y_hbm.at[srcs.at[c]], rows[b], sems[b]).wait()
        pltpu.sync_copy(rows[b], acc.at[dsts.at[c]], add=True)

        @pl.when(c + 2 < NCHUNK)
        def _():
          pltpu.async_copy(y_hbm.at[srcs.at[c + 2]], rows[b], sems[b])
      return carry

    lax.fori_loop(0, NCHUNK // 2, step, 0)
    plsc.subcore_barrier()

    pltpu.sync_copy(acc.at[pl.ds(sid * RPT, RPT)], zbuf)
    pltpu.sync_copy(zbuf, out_hbm.at[cid, pl.ds(sid * RPT, RPT)])

  return k


def _full(shape):
  return pl.BlockSpec(shape, lambda i: tuple(0 for _ in shape))


def _rows(width):
  return pl.BlockSpec((RB, width), lambda i: (i, 0))


def _tc_pre(x, Wl1, Wr1):
  def body(x_ref, wl_ref, wr_ref, ypad_ref, r_ref):
    xb = x_ref[...]
    y = jnp.dot(xb, wl_ref[...], preferred_element_type=jnp.float32)
    pad = jnp.concatenate(
        [jnp.ones((RB, 1), jnp.float32), jnp.zeros((RB, 15), jnp.float32)], 1)
    ypad_ref[...] = jnp.concatenate([y, pad], 1)
    r_ref[...] = jnp.dot(xb, wr_ref[...], preferred_element_type=jnp.float32)

  return pl.pallas_call(
      body,
      grid=(NRB,),
      in_specs=[_rows(128), _full((128, 32)), _full((128, 32))],
      out_specs=[_rows(48), _rows(32)],
      out_shape=[
          jax.ShapeDtypeStruct((N, 48), jnp.float32),
          jax.ShapeDtypeStruct((N, 32), jnp.float32),
      ],
  )(x, Wl1, Wr1)


def _tc_combine1(p, r1, bl1):
  def body(p_ref, r_ref, bl_ref, h_ref, inv_ref):
    s = p_ref[0] + p_ref[1]
    inv = 1.0 / jnp.maximum(s[:, 32:33], 1.0)
    h_ref[...] = jnp.maximum(s[:, :32] * inv + bl_ref[...] + r_ref[...], 0.0)
    inv_ref[...] = inv

  return pl.pallas_call(
      body,
      grid=(NRB,),
      in_specs=[
          pl.BlockSpec((NC, RB, 48), lambda i: (0, i, 0)),
          _rows(32), _full((1, 32)),
      ],
      out_specs=[_rows(32), _rows(1)],
      out_shape=[
          jax.ShapeDtypeStruct((N, 32), jnp.float32),
          jax.ShapeDtypeStruct((N, 1), jnp.float32),
      ],
  )(p, r1, bl1)


def _tc_layer(p, invc, h_prev, Wl, bl, Wr, fi, fo, wp, hw):
  def body(p_ref, inv_ref, h_ref, wl_ref, bl_ref, wr_ref, o_ref):
    mean = (p_ref[0, :, :fi] + p_ref[1, :, :fi]) * inv_ref[...]
    o = jnp.dot(mean, wl_ref[...], preferred_element_type=jnp.float32)
    o += jnp.dot(h_ref[:, :fi], wr_ref[...],
                 preferred_element_type=jnp.float32)
    o_ref[...] = jnp.maximum(o + bl_ref[...], 0.0)

  if fo > 64:
    def body(p_ref, inv_ref, h_ref, wl_ref, bl_ref, wr_ref, *o_refs):
      mean = (p_ref[0, :, :fi] + p_ref[1, :, :fi]) * inv_ref[...]
      o = jnp.dot(mean, wl_ref[...], preferred_element_type=jnp.float32)
      o += jnp.dot(h_ref[:, :fi], wr_ref[...],
                   preferred_element_type=jnp.float32)
      o = jnp.maximum(o + bl_ref[...], 0.0)
      for j, o_ref in enumerate(o_refs):
        o_ref[...] = o[:, j * 64:(j + 1) * 64]

    nsp = fo // 64
    return pl.pallas_call(
        body,
        grid=(NRB,),
        in_specs=[
            pl.BlockSpec((NC, RB, wp), lambda i: (0, i, 0)),
            _rows(1), _rows(hw), _full((fi, fo)), _full((1, fo)),
            _full((fi, fo)),
        ],
        out_specs=[_rows(64)] * nsp,
        out_shape=[jax.ShapeDtypeStruct((N, 64), jnp.float32)] * nsp,
    )(p, invc, h_prev, Wl, bl, Wr)

  return pl.pallas_call(
      body,
      grid=(NRB,),
      in_specs=[
          pl.BlockSpec((NC, RB, wp), lambda i: (0, i, 0)),
          _rows(1), _rows(hw), _full((fi, fo)), _full((1, fo)),
          _full((fi, fo)),
      ],
      out_specs=_rows(fo),
      out_shape=jax.ShapeDtypeStruct((N, fo), jnp.float32),
  )(p, invc, h_prev, Wl, bl, Wr)


def _tc_final(pa, pb, invc, h3a, h3b, Wl4, bl4, Wr4, batch2, W1, b1, W2, b2):
  def body(pa_ref, pb_ref, inv_ref, ha_ref, hb_ref, wl_ref, bl_ref, wr_ref,
           b_ref, w1_ref, b1_ref, w2_ref, b2_ref, o_ref, acc):
    i = pl.program_id(0)
    inv = inv_ref[...]
    mean = jnp.concatenate(
        [(pa_ref[0] + pa_ref[1]) * inv, (pb_ref[0] + pb_ref[1]) * inv], 1)
    hb = jnp.concatenate([ha_ref[...], hb_ref[...]], 1)
    h4 = jnp.dot(mean, wl_ref[...], preferred_element_type=jnp.float32)
    h4 += jnp.dot(hb, wr_ref[...], preferred_element_type=jnp.float32)
    h4 = jnp.maximum(h4 + bl_ref[...], 0.0)
    oh = (b_ref[...] == lax.broadcasted_iota(jnp.int32, (1, G), 1))
    oh = oh.astype(jnp.float32)
    h4a = jnp.concatenate([h4, jnp.ones((RB, 16), jnp.float32)], 1)
    contrib = lax.dot_general(oh, h4a, (((0,), (0,)), ((), ())),
                              preferred_element_type=jnp.float32)

    @pl.when(i == 0)
    def _():
      acc[...] = contrib

    @pl.when(i > 0)
    def _():
      acc[...] += contrib

    @pl.when(i == NRB - 1)
    def _():
      pooled = acc[:, :256] / jnp.maximum(acc[:, 256:257], 1.0)
      o1 = jnp.dot(pooled, w1_ref[...], preferred_element_type=jnp.float32)
      o1 += b1_ref[...]
      o2 = jnp.dot(o1, w2_ref[...], preferred_element_type=jnp.float32)
      o2 += b2_ref[...]
      m = jnp.max(o2, axis=1, keepdims=True)
      lse = m + jnp.log(jnp.sum(jnp.exp(o2 - m), axis=1, keepdims=True))
      o_ref[...] = o2 - lse

  return pl.pallas_call(
      body,
      grid=(NRB,),
      in_specs=[
          pl.BlockSpec((NC, RB, 64), lambda i: (0, i, 0)),
          pl.BlockSpec((NC, RB, 64), lambda i: (0, i, 0)),
          _rows(1), _rows(64), _rows(64), _full((128, 256)), _full((1, 256)),
          _full((128, 256)), _rows(1), _full((256, 128)), _full((1, 128)),
          _full((128, 32)), _full((1, 32)),
      ],
      out_specs=_full((G, 32)),
      out_shape=jax.ShapeDtypeStruct((G, 32), jnp.float32),
      scratch_shapes=[pltpu.VMEM((G, 272), jnp.float32)],
      compiler_params=pltpu.CompilerParams(fuse_transposed_lhs_in_matmul=True),
  )(pa, pb, invc, h3a, h3b, Wl4, bl4, Wr4, batch2, W1, b1, W2, b2)


def kernel(x, edge_index, batch, Wl1, bl1, Wr1, Wl2, bl2, Wr2, Wl3, bl3, Wr3,
           Wl4, bl4, Wr4, W_fc1, b_fc1, W_fc2, b_fc2):
  ei3 = edge_index.reshape(2, E // CH, CH)
  src = ei3[0]
  dst = ei3[1]
  batch2 = batch.reshape(N, 1)
  bl1r, bl2r, bl3r, bl4r = (b.reshape(1, -1) for b in (bl1, bl2, bl3, bl4))
  b1r = b_fc1.reshape(1, -1)
  b2r = b_fc2.reshape(1, -1)

  ypad, r1 = _tc_pre(x, Wl1, Wr1)
  p1 = _sc_segsum(48)(ypad, src, dst)
  h1, invc = _tc_combine1(p1, r1, bl1r)
  p2 = _sc_segsum(32)(h1, src, dst)
  h2 = _tc_layer(p2, invc, h1, Wl2, bl2r, Wr2, 32, 64, 32, 32)
  p3 = _sc_segsum(64)(h2, src, dst)
  h3a, h3b = _tc_layer(p3, invc, h2, Wl3, bl3r, Wr3, 64, 128, 64, 64)
  p4a = _sc_segsum(64)(h3a, src, dst)
  p4b = _sc_segsum(64)(h3b, src, dst)
  return _tc_final(p4a, p4b, invc, h3a, h3b, Wl4, bl4r, Wr4, batch2,
                   W_fc1, b1r, W_fc2, b2r)


# speedup vs baseline: 13.7292x; 1.0588x over previous
import functools

import jax
import jax.numpy as jnp
from jax import lax
from jax.experimental import pallas as pl
from jax.experimental.pallas import tpu as pltpu
from jax.experimental.pallas import tpu_sc as plsc

N = 10000
E = 320000
G = 16
NC, NS = 2, 16
NW = NC * NS
EPW = E // NW
CH = 125
NCHUNK = EPW // CH
NPAD = 10240
RPT = NPAD // NS
RB = 2000
NRB = N // RB


@functools.lru_cache(maxsize=None)
def _sc_segsum(dpad):
  mesh = plsc.VectorSubcoreMesh(core_axis_name="c", subcore_axis_name="s")

  @functools.partial(
      pl.kernel,
      out_type=jax.ShapeDtypeStruct((NC, NPAD, dpad), jnp.float32),
      mesh=mesh,
      compiler_params=pltpu.CompilerParams(use_tc_tiling_on_sc=False),
      scratch_types=[
          pltpu.VMEM((64, dpad), jnp.float32),
          pltpu.VMEM((CH, dpad), jnp.float32),
          pltpu.VMEM((CH, dpad), jnp.float32),
          pltpu.VMEM((NCHUNK, CH), jnp.int32),
          pltpu.VMEM((NCHUNK, CH), jnp.int32),
          pltpu.VMEM_SHARED((NPAD, dpad), jnp.float32),
          pltpu.SemaphoreType.DMA,
          pltpu.SemaphoreType.DMA,
      ],
  )
  def k(y_hbm, eidx_hbm, out_hbm, zbuf, rows0, rows1, srcs, dsts,
        acc, sem0, sem1):
    cid = lax.axis_index("c")
    sid = lax.axis_index("s")
    wid = cid * NS + sid
    rows = (rows0, rows1)
    sems = (sem0, sem1)

    def zrow(r, carry):
      for j in range(dpad // 16):
        zbuf[r, pl.ds(j * 16, 16)] = jnp.zeros((16,), jnp.float32)
      return carry

    lax.fori_loop(0, 64, zrow, 0)
    for t in range(RPT // 64):
      pltpu.sync_copy(zbuf, acc.at[pl.ds(sid * RPT + t * 64, 64)])

    cbase = wid * NCHUNK
    pltpu.sync_copy(eidx_hbm.at[0, pl.ds(cbase, NCHUNK)], srcs)
    pltpu.sync_copy(eidx_hbm.at[1, pl.ds(cbase, NCHUNK)], dsts)
    plsc.subcore_barrier()

    pltpu.async_copy(y_hbm.at[srcs.at[0]], rows0, sem0)
    pltpu.async_copy(y_hbm.at[srcs.at[1]], rows1, sem1)

    def step(kk, carry):
      for b in range(2):
        c = 2 * kk + b
        pltpu.make_async_copy(y_hbm.at[srcs.at[c]], rows[b], sems[b]).wait()
        pltpu.sync_copy(rows[b], acc.at[dsts.at[c]], add=True)

        @pl.when(c + 2 < NCHUNK)
        def _():
          pltpu.async_copy(y_hbm.at[srcs.at[c + 2]], rows[b], sems[b])
      return carry

    lax.fori_loop(0, NCHUNK // 2, step, 0)
    plsc.subcore_barrier()

    pltpu.sync_copy(acc.at[pl.ds(sid * RPT, RPT)],
                    out_hbm.at[cid, pl.ds(sid * RPT, RPT)])

  return k


def _full(shape):
  return pl.BlockSpec(shape, lambda i: tuple(0 for _ in shape))


def _rows(width):
  return pl.BlockSpec((RB, width), lambda i: (i, 0))


def _tc_pre(x, Wl1, Wr1):
  def body(x_ref, wl_ref, wr_ref, ypad_ref, r_ref):
    xb = x_ref[...]
    y = jnp.dot(xb, wl_ref[...], preferred_element_type=jnp.float32)
    pad = jnp.concatenate(
        [jnp.ones((RB, 1), jnp.float32), jnp.zeros((RB, 15), jnp.float32)], 1)
    ypad_ref[...] = jnp.concatenate([y, pad], 1)
    r_ref[...] = jnp.dot(xb, wr_ref[...], preferred_element_type=jnp.float32)

  return pl.pallas_call(
      body,
      grid=(NRB,),
      in_specs=[_rows(128), _full((128, 32)), _full((128, 32))],
      out_specs=[_rows(48), _rows(32)],
      out_shape=[
          jax.ShapeDtypeStruct((N, 48), jnp.float32),
          jax.ShapeDtypeStruct((N, 32), jnp.float32),
      ],
  )(x, Wl1, Wr1)


def _tc_combine1(p, r1, bl1):
  def body(p_ref, r_ref, bl_ref, h_ref, inv_ref):
    s = p_ref[0] + p_ref[1]
    inv = 1.0 / jnp.maximum(s[:, 32:33], 1.0)
    h_ref[...] = jnp.maximum(s[:, :32] * inv + bl_ref[...] + r_ref[...], 0.0)
    inv_ref[...] = inv

  return pl.pallas_call(
      body,
      grid=(NRB,),
      in_specs=[
          pl.BlockSpec((NC, RB, 48), lambda i: (0, i, 0)),
          _rows(32), _full((1, 32)),
      ],
      out_specs=[_rows(32), _rows(1)],
      out_shape=[
          jax.ShapeDtypeStruct((N, 32), jnp.float32),
          jax.ShapeDtypeStruct((N, 1), jnp.float32),
      ],
  )(p, r1, bl1)


def _tc_layer(p, invc, h_prev, Wl, bl, Wr, fi, fo, wp, hw):
  def body(p_ref, inv_ref, h_ref, wl_ref, bl_ref, wr_ref, o_ref):
    mean = (p_ref[0, :, :fi] + p_ref[1, :, :fi]) * inv_ref[...]
    o = jnp.dot(mean, wl_ref[...], preferred_element_type=jnp.float32)
    o += jnp.dot(h_ref[:, :fi], wr_ref[...],
                 preferred_element_type=jnp.float32)
    o_ref[...] = jnp.maximum(o + bl_ref[...], 0.0)

  if fo > 64:
    def body(p_ref, inv_ref, h_ref, wl_ref, bl_ref, wr_ref, *o_refs):
      mean = (p_ref[0, :, :fi] + p_ref[1, :, :fi]) * inv_ref[...]
      o = jnp.dot(mean, wl_ref[...], preferred_element_type=jnp.float32)
      o += jnp.dot(h_ref[:, :fi], wr_ref[...],
                   preferred_element_type=jnp.float32)
      o = jnp.maximum(o + bl_ref[...], 0.0)
      for j, o_ref in enumerate(o_refs):
        o_ref[...] = o[:, j * 64:(j + 1) * 64]

    nsp = fo // 64
    return pl.pallas_call(
        body,
        grid=(NRB,),
        in_specs=[
            pl.BlockSpec((NC, RB, wp), lambda i: (0, i, 0)),
            _rows(1), _rows(hw), _full((fi, fo)), _full((1, fo)),
            _full((fi, fo)),
        ],
        out_specs=[_rows(64)] * nsp,
        out_shape=[jax.ShapeDtypeStruct((N, 64), jnp.float32)] * nsp,
    )(p, invc, h_prev, Wl, bl, Wr)

  return pl.pallas_call(
      body,
      grid=(NRB,),
      in_specs=[
          pl.BlockSpec((NC, RB, wp), lambda i: (0, i, 0)),
          _rows(1), _rows(hw), _full((fi, fo)), _full((1, fo)),
          _full((fi, fo)),
      ],
      out_specs=_rows(fo),
      out_shape=jax.ShapeDtypeStruct((N, fo), jnp.float32),
  )(p, invc, h_prev, Wl, bl, Wr)


def _tc_final(pa, pb, invc, h3a, h3b, Wl4, bl4, Wr4, batch2, W1, b1, W2, b2):
  def body(pa_ref, pb_ref, inv_ref, ha_ref, hb_ref, wl_ref, bl_ref, wr_ref,
           b_ref, w1_ref, b1_ref, w2_ref, b2_ref, o_ref, acc):
    i = pl.program_id(0)
    inv = inv_ref[...]
    mean = jnp.concatenate(
        [(pa_ref[0] + pa_ref[1]) * inv, (pb_ref[0] + pb_ref[1]) * inv], 1)
    hb = jnp.concatenate([ha_ref[...], hb_ref[...]], 1)
    h4 = jnp.dot(mean, wl_ref[...], preferred_element_type=jnp.float32)
    h4 += jnp.dot(hb, wr_ref[...], preferred_element_type=jnp.float32)
    h4 = jnp.maximum(h4 + bl_ref[...], 0.0)
    oh = (b_ref[...] == lax.broadcasted_iota(jnp.int32, (1, G), 1))
    oh = oh.astype(jnp.float32)
    h4a = jnp.concatenate([h4, jnp.ones((RB, 16), jnp.float32)], 1)
    contrib = lax.dot_general(oh, h4a, (((0,), (0,)), ((), ())),
                              preferred_element_type=jnp.float32)

    @pl.when(i == 0)
    def _():
      acc[...] = contrib

    @pl.when(i > 0)
    def _():
      acc[...] += contrib

    @pl.when(i == NRB - 1)
    def _():
      pooled = acc[:, :256] / jnp.maximum(acc[:, 256:257], 1.0)
      o1 = jnp.dot(pooled, w1_ref[...], preferred_element_type=jnp.float32)
      o1 += b1_ref[...]
      o2 = jnp.dot(o1, w2_ref[...], preferred_element_type=jnp.float32)
      o2 += b2_ref[...]
      m = jnp.max(o2, axis=1, keepdims=True)
      lse = m + jnp.log(jnp.sum(jnp.exp(o2 - m), axis=1, keepdims=True))
      o_ref[...] = o2 - lse

  return pl.pallas_call(
      body,
      grid=(NRB,),
      in_specs=[
          pl.BlockSpec((NC, RB, 64), lambda i: (0, i, 0)),
          pl.BlockSpec((NC, RB, 64), lambda i: (0, i, 0)),
          _rows(1), _rows(64), _rows(64), _full((128, 256)), _full((1, 256)),
          _full((128, 256)), _rows(1), _full((256, 128)), _full((1, 128)),
          _full((128, 32)), _full((1, 32)),
      ],
      out_specs=_full((G, 32)),
      out_shape=jax.ShapeDtypeStruct((G, 32), jnp.float32),
      scratch_shapes=[pltpu.VMEM((G, 272), jnp.float32)],
      compiler_params=pltpu.CompilerParams(fuse_transposed_lhs_in_matmul=True),
  )(pa, pb, invc, h3a, h3b, Wl4, bl4, Wr4, batch2, W1, b1, W2, b2)


def kernel(x, edge_index, batch, Wl1, bl1, Wr1, Wl2, bl2, Wr2, Wl3, bl3, Wr3,
           Wl4, bl4, Wr4, W_fc1, b_fc1, W_fc2, b_fc2):
  ei3 = edge_index.reshape(2, E // CH, CH)
  batch2 = batch.reshape(N, 1)
  bl1r, bl2r, bl3r, bl4r = (b.reshape(1, -1) for b in (bl1, bl2, bl3, bl4))
  b1r = b_fc1.reshape(1, -1)
  b2r = b_fc2.reshape(1, -1)

  ypad, r1 = _tc_pre(x, Wl1, Wr1)
  p1 = _sc_segsum(48)(ypad, ei3)
  h1, invc = _tc_combine1(p1, r1, bl1r)
  p2 = _sc_segsum(32)(h1, ei3)
  h2 = _tc_layer(p2, invc, h1, Wl2, bl2r, Wr2, 32, 64, 32, 32)
  p3 = _sc_segsum(64)(h2, ei3)
  h3a, h3b = _tc_layer(p3, invc, h2, Wl3, bl3r, Wr3, 64, 128, 64, 64)
  p4a = _sc_segsum(64)(h3a, ei3)
  p4b = _sc_segsum(64)(h3b, ei3)
  return _tc_final(p4a, p4b, invc, h3a, h3b, Wl4, bl4r, Wr4, batch2,
                   W_fc1, b1r, W_fc2, b2r)
